# baseline (speedup 1.0000x reference)
import functools

import jax
import jax.numpy as jnp
from jax import lax
from jax.experimental import pallas as pl
from jax.experimental.pallas import tpu as pltpu

_B = 2
_N = 8192
_C = 128
_NPOINT = 2048
_NSAMPLE = 32
_ROWS = _N // 128


def _fps_body(npoint, x_ref, idx_ref, cx_ref, cy_ref, cz_ref):
    iota = (lax.broadcasted_iota(jnp.int32, (_ROWS, 128), 0) * 128
            + lax.broadcasted_iota(jnp.int32, (_ROWS, 128), 1))
    xs = [[x_ref[b, c] for c in range(3)] for b in range(_B)]

    def body(i, carry):
        fars, dists = carry
        new_fars = []
        new_dists = []
        for b in range(_B):
            far = fars[b]
            x, y, z = xs[b]
            mask = iota == far
            cx = jnp.sum(jnp.where(mask, x, 0.0))
            cy = jnp.sum(jnp.where(mask, y, 0.0))
            cz = jnp.sum(jnp.where(mask, z, 0.0))
            idx_ref[b, pl.ds(i, 1), :] = jnp.broadcast_to(far, (1, 1))
            cx_ref[b, pl.ds(i, 1), :] = jnp.broadcast_to(cx, (1, 1))
            cy_ref[b, pl.ds(i, 1), :] = jnp.broadcast_to(cy, (1, 1))
            cz_ref[b, pl.ds(i, 1), :] = jnp.broadcast_to(cz, (1, 1))
            d = (x - cx) ** 2 + (y - cy) ** 2 + (z - cz) ** 2
            nd = jnp.minimum(dists[b], d)
            m = jnp.max(nd)
            cand = jnp.where(nd == m, iota, jnp.int32(2**31 - 1))
            nf = jnp.min(cand)
            new_fars.append(nf)
            new_dists.append(nd)
        return tuple(new_fars), tuple(new_dists)

    far0 = jnp.int32(0)
    d0 = jnp.full((_ROWS, 128), 1e10, jnp.float32)
    lax.fori_loop(0, npoint, body, ((far0, far0), (d0, d0)))


def _fps_pallas(xt, npoint):
    out_shapes = (
        jax.ShapeDtypeStruct((_B, npoint, 1), jnp.int32),
        jax.ShapeDtypeStruct((_B, npoint, 1), jnp.float32),
        jax.ShapeDtypeStruct((_B, npoint, 1), jnp.float32),
        jax.ShapeDtypeStruct((_B, npoint, 1), jnp.float32),
    )
    return pl.pallas_call(
        functools.partial(_fps_body, npoint),
        out_shape=out_shapes,
    )(xt)


def _mlp_body(ch, g_ref, w1_ref, b1_ref, w2_ref, b2_ref, o_ref):
    g = g_ref[0]
    h = jnp.dot(g, w1_ref[...], preferred_element_type=jnp.float32)
    h = jnp.maximum(h + b1_ref[...], 0.0)
    h = jnp.dot(h, w2_ref[...], preferred_element_type=jnp.float32)
    h = jnp.maximum(h + b2_ref[...], 0.0)
    o_ref[0] = jnp.max(h.reshape(ch, _NSAMPLE, 256), axis=1)


def _mlp_pallas(g, W1, b1, W2, b2):
    ch = 128
    grid = (_B, _NPOINT // ch)
    return pl.pallas_call(
        functools.partial(_mlp_body, ch),
        grid=grid,
        in_specs=[
            pl.BlockSpec((1, ch * _NSAMPLE, 131), lambda b, c: (b, c, 0)),
            pl.BlockSpec((131, 128), lambda b, c: (0, 0)),
            pl.BlockSpec((1, 128), lambda b, c: (0, 0)),
            pl.BlockSpec((128, 256), lambda b, c: (0, 0)),
            pl.BlockSpec((1, 256), lambda b, c: (0, 0)),
        ],
        out_specs=pl.BlockSpec((1, ch, 256), lambda b, c: (b, c, 0)),
        out_shape=jax.ShapeDtypeStruct((_B, _NPOINT, 256), jnp.float32),
    )(g, W1, b1.reshape(1, 128), W2, b2.reshape(1, 256))


def kernel(xyz, features, W1, b1, W2, b2):
    xt = xyz.transpose(0, 2, 1).reshape(_B, 3, _ROWS, 128)
    idx, cx, cy, cz = _fps_pallas(xt, _NPOINT)
    new_xyz = jnp.concatenate([cx, cy, cz], axis=-1)

    if True:
        def _knn(xyz_b, new_xyz_b):
            d2 = (jnp.sum(new_xyz_b ** 2, axis=-1)[:, None]
                  - 2.0 * (new_xyz_b @ xyz_b.T)
                  + jnp.sum(xyz_b ** 2, axis=-1)[None, :])
            _, nidx = lax.top_k(-d2, _NSAMPLE)
            return nidx
        nidx = jax.vmap(_knn)(xyz, new_xyz)
        return new_xyz, jnp.broadcast_to(jnp.sum(nidx, axis=-1).astype(jnp.float32)[..., None], (_B, _NPOINT, 256))
    def _group(xyz_b, feat_b, new_xyz_b):
        d2 = (jnp.sum(new_xyz_b ** 2, axis=-1)[:, None]
              - 2.0 * (new_xyz_b @ xyz_b.T)
              + jnp.sum(xyz_b ** 2, axis=-1)[None, :])
        _, nidx = lax.top_k(-d2, _NSAMPLE)
        grouped_xyz = xyz_b[nidx] - new_xyz_b[:, None, :]
        grouped_feat = feat_b[nidx]
        return jnp.concatenate([grouped_xyz, grouped_feat], axis=-1)

    g = jax.vmap(_group)(xyz, features, new_xyz)
    g = g.reshape(_B, _NPOINT * _NSAMPLE, _C + 3)

    new_feat = _mlp_pallas(g, W1, b1, W2, b2)
    return new_xyz, new_feat


# speedup vs baseline: 1.2484x; 1.2484x over previous
import functools

import jax
import jax.numpy as jnp
from jax import lax
from jax.experimental import pallas as pl
from jax.experimental.pallas import tpu as pltpu

_B = 2
_N = 8192
_C = 128
_NPOINT = 2048
_NSAMPLE = 32
_ROWS = _N // 128


def _fps_body(npoint, x_ref, idx_ref, cx_ref, cy_ref, cz_ref):
    iota = (lax.broadcasted_iota(jnp.int32, (_ROWS, 128), 0) * 128
            + lax.broadcasted_iota(jnp.int32, (_ROWS, 128), 1))
    xs = [[x_ref[b, c] for c in range(3)] for b in range(_B)]

    def body(i, carry):
        fars, dists = carry
        new_fars = []
        new_dists = []
        for b in range(_B):
            far = fars[b]
            x, y, z = xs[b]
            mask = iota == far
            cx = jnp.sum(jnp.where(mask, x, 0.0))
            cy = jnp.sum(jnp.where(mask, y, 0.0))
            cz = jnp.sum(jnp.where(mask, z, 0.0))
            idx_ref[b, pl.ds(i, 1), :] = jnp.broadcast_to(far, (1, 1))
            cx_ref[b, pl.ds(i, 1), :] = jnp.broadcast_to(cx, (1, 1))
            cy_ref[b, pl.ds(i, 1), :] = jnp.broadcast_to(cy, (1, 1))
            cz_ref[b, pl.ds(i, 1), :] = jnp.broadcast_to(cz, (1, 1))
            d = (x - cx) ** 2 + (y - cy) ** 2 + (z - cz) ** 2
            nd = jnp.minimum(dists[b], d)
            m = jnp.max(nd)
            cand = jnp.where(nd == m, iota, jnp.int32(2**31 - 1))
            nf = jnp.min(cand)
            new_fars.append(nf)
            new_dists.append(nd)
        return tuple(new_fars), tuple(new_dists)

    far0 = jnp.int32(0)
    d0 = jnp.full((_ROWS, 128), 1e10, jnp.float32)
    lax.fori_loop(0, npoint, body, ((far0, far0), (d0, d0)))


def _fps_pallas(xt, npoint):
    out_shapes = (
        jax.ShapeDtypeStruct((_B, npoint, 1), jnp.int32),
        jax.ShapeDtypeStruct((_B, npoint, 1), jnp.float32),
        jax.ShapeDtypeStruct((_B, npoint, 1), jnp.float32),
        jax.ShapeDtypeStruct((_B, npoint, 1), jnp.float32),
    )
    return pl.pallas_call(
        functools.partial(_fps_body, npoint),
        out_shape=out_shapes,
    )(xt)


def _knn_body(cen_ref, cn_ref, xyzt_ref, pn_ref, o_ref):
    a = cen_ref[0]
    bm = xyzt_ref[0]
    d2 = (cn_ref[0]
          - 2.0 * jnp.dot(a, bm, preferred_element_type=jnp.float32)
          + pn_ref[0])
    iota = lax.broadcasted_iota(jnp.int32, (8, _N), 1)
    big = jnp.float32(3e38)
    bigi = jnp.int32(2**31 - 1)
    cols = []
    for _ in range(_NSAMPLE):
        m = jnp.min(d2, axis=1, keepdims=True)
        eq = d2 == m
        j = jnp.min(jnp.where(eq, iota, bigi), axis=1, keepdims=True)
        cols.append(j)
        d2 = jnp.where(eq, big, d2)
    o_ref[0] = jnp.concatenate(cols, axis=1)


def _knn_pallas(new_xyz, xyz):
    cen8 = jnp.concatenate(
        [new_xyz, jnp.zeros((_B, _NPOINT, 5), jnp.float32)], axis=-1)
    cn = jnp.sum(new_xyz ** 2, axis=-1, keepdims=True)
    xyzt = jnp.concatenate(
        [xyz.transpose(0, 2, 1), jnp.zeros((_B, 5, _N), jnp.float32)], axis=1)
    pn = jnp.sum(xyz ** 2, axis=-1)[:, None, :]
    grid = (_B, _NPOINT // 8)
    return pl.pallas_call(
        _knn_body,
        grid=grid,
        in_specs=[
            pl.BlockSpec((1, 8, 8), lambda b, c: (b, c, 0)),
            pl.BlockSpec((1, 8, 1), lambda b, c: (b, c, 0)),
            pl.BlockSpec((1, 8, _N), lambda b, c: (b, 0, 0)),
            pl.BlockSpec((1, 1, _N), lambda b, c: (b, 0, 0)),
        ],
        out_specs=pl.BlockSpec((1, 8, _NSAMPLE), lambda b, c: (b, c, 0)),
        out_shape=jax.ShapeDtypeStruct((_B, _NPOINT, _NSAMPLE), jnp.int32),
    )(cen8, cn, xyzt, pn)


def _mlp_body(ch, g_ref, w1_ref, b1_ref, w2_ref, b2_ref, o_ref):
    g = g_ref[0]
    h = jnp.dot(g, w1_ref[...], preferred_element_type=jnp.float32)
    h = jnp.maximum(h + b1_ref[...], 0.0)
    h = jnp.dot(h, w2_ref[...], preferred_element_type=jnp.float32)
    h = jnp.maximum(h + b2_ref[...], 0.0)
    o_ref[0] = jnp.max(h.reshape(ch, _NSAMPLE, 256), axis=1)


def _mlp_pallas(g, W1, b1, W2, b2):
    ch = 128
    grid = (_B, _NPOINT // ch)
    return pl.pallas_call(
        functools.partial(_mlp_body, ch),
        grid=grid,
        in_specs=[
            pl.BlockSpec((1, ch * _NSAMPLE, 131), lambda b, c: (b, c, 0)),
            pl.BlockSpec((131, 128), lambda b, c: (0, 0)),
            pl.BlockSpec((1, 128), lambda b, c: (0, 0)),
            pl.BlockSpec((128, 256), lambda b, c: (0, 0)),
            pl.BlockSpec((1, 256), lambda b, c: (0, 0)),
        ],
        out_specs=pl.BlockSpec((1, ch, 256), lambda b, c: (b, c, 0)),
        out_shape=jax.ShapeDtypeStruct((_B, _NPOINT, 256), jnp.float32),
    )(g, W1, b1.reshape(1, 128), W2, b2.reshape(1, 256))


def kernel(xyz, features, W1, b1, W2, b2):
    xt = xyz.transpose(0, 2, 1).reshape(_B, 3, _ROWS, 128)
    idx, cx, cy, cz = _fps_pallas(xt, _NPOINT)
    new_xyz = jnp.concatenate([cx, cy, cz], axis=-1)

    nidx = _knn_pallas(new_xyz, xyz)

    def _group(xyz_b, feat_b, new_xyz_b, nidx_b):
        grouped_xyz = xyz_b[nidx_b] - new_xyz_b[:, None, :]
        grouped_feat = feat_b[nidx_b]
        return jnp.concatenate([grouped_xyz, grouped_feat], axis=-1)

    g = jax.vmap(_group)(xyz, features, new_xyz, nidx)
    g = g.reshape(_B, _NPOINT * _NSAMPLE, _C + 3)

    new_feat = _mlp_pallas(g, W1, b1, W2, b2)
    return new_xyz, new_feat


# speedup vs baseline: 2.6295x; 2.1064x over previous
import functools

import jax
import jax.numpy as jnp
from jax import lax
from jax.experimental import pallas as pl
from jax.experimental.pallas import tpu as pltpu
from jax.experimental.pallas import tpu_sc as plsc

_B = 2
_N = 8192
_C = 128
_NPOINT = 2048
_NSAMPLE = 32
_ROWS = _N // 128


def _fps_body(npoint, x_ref, idx_ref, cx_ref, cy_ref, cz_ref):
    iota = (lax.broadcasted_iota(jnp.int32, (_ROWS, 128), 0) * 128
            + lax.broadcasted_iota(jnp.int32, (_ROWS, 128), 1))
    xs = [[x_ref[b, c] for c in range(3)] for b in range(_B)]

    def body(i, carry):
        fars, dists = carry
        new_fars = []
        new_dists = []
        for b in range(_B):
            far = fars[b]
            x, y, z = xs[b]
            mask = iota == far
            cx = jnp.sum(jnp.where(mask, x, 0.0))
            cy = jnp.sum(jnp.where(mask, y, 0.0))
            cz = jnp.sum(jnp.where(mask, z, 0.0))
            idx_ref[b, pl.ds(i, 1), :] = jnp.broadcast_to(far, (1, 1))
            cx_ref[b, pl.ds(i, 1), :] = jnp.broadcast_to(cx, (1, 1))
            cy_ref[b, pl.ds(i, 1), :] = jnp.broadcast_to(cy, (1, 1))
            cz_ref[b, pl.ds(i, 1), :] = jnp.broadcast_to(cz, (1, 1))
            d = (x - cx) ** 2 + (y - cy) ** 2 + (z - cz) ** 2
            nd = jnp.minimum(dists[b], d)
            m = jnp.max(nd)
            cand = jnp.where(nd == m, iota, jnp.int32(2**31 - 1))
            nf = jnp.min(cand)
            new_fars.append(nf)
            new_dists.append(nd)
        return tuple(new_fars), tuple(new_dists)

    far0 = jnp.int32(0)
    d0 = jnp.full((_ROWS, 128), 1e10, jnp.float32)
    lax.fori_loop(0, npoint, body, ((far0, far0), (d0, d0)))


def _fps_pallas(xt, npoint):
    out_shapes = (
        jax.ShapeDtypeStruct((_B, npoint, 1), jnp.int32),
        jax.ShapeDtypeStruct((_B, npoint, 1), jnp.float32),
        jax.ShapeDtypeStruct((_B, npoint, 1), jnp.float32),
        jax.ShapeDtypeStruct((_B, npoint, 1), jnp.float32),
    )
    return pl.pallas_call(
        functools.partial(_fps_body, npoint),
        out_shape=out_shapes,
    )(xt)


def _knn_body(cen_ref, cn_ref, xyzt_ref, pn_ref, o_ref):
    a = cen_ref[0]
    bm = xyzt_ref[0]
    d2 = (cn_ref[0]
          - 2.0 * jnp.dot(a, bm, preferred_element_type=jnp.float32)
          + pn_ref[0])
    iota = lax.broadcasted_iota(jnp.int32, (8, _N), 1)
    big = jnp.float32(3e38)
    bigi = jnp.int32(2**31 - 1)
    cols = []
    for _ in range(_NSAMPLE):
        m = jnp.min(d2, axis=1, keepdims=True)
        eq = d2 == m
        j = jnp.min(jnp.where(eq, iota, bigi), axis=1, keepdims=True)
        cols.append(j)
        d2 = jnp.where(eq, big, d2)
    o_ref[0] = jnp.concatenate(cols, axis=1)


def _knn_pallas(new_xyz, xyz):
    cen8 = jnp.concatenate(
        [new_xyz, jnp.zeros((_B, _NPOINT, 5), jnp.float32)], axis=-1)
    cn = jnp.sum(new_xyz ** 2, axis=-1, keepdims=True)
    xyzt = jnp.concatenate(
        [xyz.transpose(0, 2, 1), jnp.zeros((_B, 5, _N), jnp.float32)], axis=1)
    pn = jnp.sum(xyz ** 2, axis=-1)[:, None, :]
    grid = (_B, _NPOINT // 8)
    return pl.pallas_call(
        _knn_body,
        grid=grid,
        in_specs=[
            pl.BlockSpec((1, 8, 8), lambda b, c: (b, c, 0)),
            pl.BlockSpec((1, 8, 1), lambda b, c: (b, c, 0)),
            pl.BlockSpec((1, 8, _N), lambda b, c: (b, 0, 0)),
            pl.BlockSpec((1, 1, _N), lambda b, c: (b, 0, 0)),
        ],
        out_specs=pl.BlockSpec((1, 8, _NSAMPLE), lambda b, c: (b, c, 0)),
        out_shape=jax.ShapeDtypeStruct((_B, _NPOINT, _NSAMPLE), jnp.int32),
    )(cen8, cn, xyzt, pn)


def _pmat_body(x_ref, w1_ref, o_ref):
    o_ref[0] = jnp.dot(x_ref[0], w1_ref[...],
                       preferred_element_type=jnp.float32)


def _pmat_pallas(x131, W1):
    rows = 1024
    grid = (_B, _N // rows)
    return pl.pallas_call(
        _pmat_body,
        grid=grid,
        in_specs=[
            pl.BlockSpec((1, rows, _C + 3), lambda b, c: (b, c, 0)),
            pl.BlockSpec((_C + 3, _C), lambda b, c: (0, 0)),
        ],
        out_specs=pl.BlockSpec((1, rows, _C), lambda b, c: (b, c, 0)),
        out_shape=jax.ShapeDtypeStruct((_B, _N, _C), jnp.float32),
    )(x131, W1)


_GROWS = _B * _NPOINT * _NSAMPLE
_NW = 32
_RPW = _GROWS // _NW
_GCHUNK = 512
_GNCH = _RPW // _GCHUNK


def _gather_body(p_hbm, idx_hbm, out_hbm, idx_v, rows_v, sem):
    wid = lax.axis_index("s") * 2 + lax.axis_index("c")
    base = wid * _RPW

    def chunk(k, carry):
        off = pl.multiple_of(base + k * _GCHUNK, _GCHUNK)
        pltpu.sync_copy(idx_hbm.at[pl.ds(off, _GCHUNK)], idx_v)
        pltpu.async_copy(p_hbm.at[idx_v], rows_v, sem).wait()
        pltpu.sync_copy(rows_v, out_hbm.at[pl.ds(off, _GCHUNK)])
        return carry

    lax.fori_loop(0, _GNCH, chunk, 0)


def _gather_pallas(p_flat, flat_idx):
    mesh = plsc.VectorSubcoreMesh(core_axis_name="c", subcore_axis_name="s")
    return pl.kernel(
        _gather_body,
        out_type=jax.ShapeDtypeStruct((_GROWS, _C), jnp.float32),
        mesh=mesh,
        scratch_types=[
            pltpu.VMEM((_GCHUNK,), jnp.int32),
            pltpu.VMEM((_GCHUNK, _C), jnp.float32),
            pltpu.SemaphoreType.DMA,
        ],
    )(p_flat, flat_idx)


def _mlp_body(ch, g_ref, cen8_ref, w1a_ref, b1_ref, w2_ref, b2_ref, o_ref):
    corr = jnp.dot(cen8_ref[...], w1a_ref[...],
                   preferred_element_type=jnp.float32)
    t = b1_ref[...] - corr
    h = g_ref[...].reshape(ch, _NSAMPLE, _C) + t[:, None, :]
    h = jnp.maximum(h, 0.0).reshape(ch * _NSAMPLE, _C)
    h = jnp.dot(h, w2_ref[...], preferred_element_type=jnp.float32)
    h = jnp.maximum(h + b2_ref[...], 0.0)
    o_ref[...] = jnp.max(h.reshape(ch, _NSAMPLE, 256), axis=1)


def _mlp_pallas(g, cen8, W1, b1, W2, b2):
    ch = 128
    grid = (_B * _NPOINT // ch,)
    w1a8 = jnp.concatenate(
        [W1[:3], jnp.zeros((5, _C), jnp.float32)], axis=0)
    return pl.pallas_call(
        functools.partial(_mlp_body, ch),
        grid=grid,
        in_specs=[
            pl.BlockSpec((ch * _NSAMPLE, _C), lambda c: (c, 0)),
            pl.BlockSpec((ch, 8), lambda c: (c, 0)),
            pl.BlockSpec((8, _C), lambda c: (0, 0)),
            pl.BlockSpec((1, _C), lambda c: (0, 0)),
            pl.BlockSpec((_C, 256), lambda c: (0, 0)),
            pl.BlockSpec((1, 256), lambda c: (0, 0)),
        ],
        out_specs=pl.BlockSpec((ch, 256), lambda c: (c, 0)),
        out_shape=jax.ShapeDtypeStruct((_B * _NPOINT, 256), jnp.float32),
    )(g, cen8, w1a8, b1.reshape(1, _C), W2, b2.reshape(1, 256))


def kernel(xyz, features, W1, b1, W2, b2):
    xt = xyz.transpose(0, 2, 1).reshape(_B, 3, _ROWS, 128)
    idx, cx, cy, cz = _fps_pallas(xt, _NPOINT)
    new_xyz = jnp.concatenate([cx, cy, cz], axis=-1)

    nidx = _knn_pallas(new_xyz, xyz)

    x131 = jnp.concatenate([xyz, features], axis=-1)
    p = _pmat_pallas(x131, W1).reshape(_B * _N, _C)

    flat_idx = (nidx + (jnp.arange(_B, dtype=jnp.int32) * _N)[:, None, None])
    g = _gather_pallas(p, flat_idx.reshape(_GROWS))

    cen8 = jnp.concatenate(
        [new_xyz, jnp.zeros((_B, _NPOINT, 5), jnp.float32)], axis=-1)
    new_feat = _mlp_pallas(g, cen8.reshape(_B * _NPOINT, 8), W1, b1, W2, b2)
    return new_xyz, new_feat.reshape(_B, _NPOINT, 256)
